# baseline (speedup 1.0000x reference)
import functools

import jax
import jax.numpy as jnp
from jax import lax
from jax.experimental import pallas as pl
from jax.experimental.pallas import tpu as pltpu
from jax.experimental.pallas import tpu_sc as plsc

_B, _H, _W, _C = 4, 384, 384, 96
_F = 16
_HO = (_H - _F) // _F + 1
_WO = (_W - _F) // _F + 1
_U = _B * _HO
_ROW = _F * _C
_C_OUT = _F * _F * _C
_L = 16

_NW = 32
_UPW = _U // _NW
_UNITS = _UPW * _WO
_NS = 2


def _fold_body(x_hbm, out_hbm, buf_a, buf_b, in_a, in_b, out_a, out_b):
    in_sems = (in_a, in_b)
    out_sems = (out_a, out_b)
    wid = lax.axis_index("s") * 2 + lax.axis_index("c")

    def unit_idx(t):
        du = t // _WO
        fx = t % _WO
        u = wid * _UPW + du
        b = u // _HO
        i = u % _HO
        return b, i, fx

    def fire_gather(g, s):
        b, i, fx = unit_idx(g)
        pltpu.async_copy(
            x_hbm.at[b, pl.ds(i * _F, _F), pl.ds(fx * _F, _F), :],
            buf_a.at[s], in_sems[s])

    def wait_gather(s):
        pltpu.make_async_copy(
            x_hbm.at[0, pl.ds(0, _F), pl.ds(0, _F), :],
            buf_a.at[s], in_sems[s]).wait()

    def repack(s):
        def row(yi, _):
            base = yi * _ROW
            for xi in range(_F):
                for c in range(_C // _L):
                    buf_b[s, pl.ds(base + xi * _C + c * _L, _L)] = (
                        buf_a[s, yi, xi, pl.ds(c * _L, _L)])
            return ()
        lax.fori_loop(0, _F, row, (), unroll=2)

    def fire_write(g, s):
        b, i, fx = unit_idx(g)
        pltpu.async_copy(buf_b.at[s], out_hbm.at[b, i, fx, :], out_sems[s])

    def wait_write(s):
        pltpu.make_async_copy(
            buf_b.at[s], out_hbm.at[0, 0, 0, :], out_sems[s]).wait()

    fire_gather(jnp.int32(0), 0)

    def iter_body(t, _):
        for s in range(_NS):
            g = t * _NS + s

            @pl.when(g + 1 < _UNITS)
            def _prefetch():
                fire_gather(g + 1, 1 - s)

            wait_gather(s)

            @pl.when(g >= _NS)
            def _recycle():
                wait_write(s)

            repack(s)
            fire_write(g, s)
        return ()

    lax.fori_loop(0, _UNITS // _NS, iter_body, ())
    for s in range(_NS):
        wait_write(s)


_fold = functools.partial(
    pl.kernel,
    mesh=plsc.VectorSubcoreMesh(core_axis_name="c", subcore_axis_name="s"),
    out_type=jax.ShapeDtypeStruct((_B, _WO, _HO, _C_OUT), jnp.float32),
    scratch_types=[
        pltpu.VMEM((_NS, _F, _F, _C), jnp.float32),
        pltpu.VMEM((_NS, _C_OUT), jnp.float32),
        pltpu.SemaphoreType.DMA,
        pltpu.SemaphoreType.DMA,
        pltpu.SemaphoreType.DMA,
        pltpu.SemaphoreType.DMA,
    ],
)(_fold_body)


def kernel(tensor):
    return _fold(tensor)


# speedup vs baseline: 1.3549x; 1.3549x over previous
import functools

import jax
import jax.numpy as jnp
from jax import lax
from jax.experimental import pallas as pl
from jax.experimental.pallas import tpu as pltpu
from jax.experimental.pallas import tpu_sc as plsc

_B, _H, _W, _C = 4, 384, 384, 96
_F = 16
_HO = (_H - _F) // _F + 1
_WO = (_W - _F) // _F + 1
_U = _B * _HO
_ROW = _F * _C
_C_OUT = _F * _F * _C
_L = 16

_NW = 32
_UPW = _U // _NW
_UNITS = _UPW * _WO
_NS = 2


def _fold_body(x_hbm, out_hbm, buf_a, buf_b, in_a, in_b, out_a, out_b):
    in_sems = (in_a, in_b)
    out_sems = (out_a, out_b)
    wid = lax.axis_index("s") * 2 + lax.axis_index("c")

    def unit_idx(t):
        du = t // _WO
        fx = t % _WO
        u = wid * _UPW + du
        b = u // _HO
        i = u % _HO
        return b, i, fx

    def fire_gather(g, s):
        b, i, fx = unit_idx(g)
        pltpu.async_copy(
            x_hbm.at[b, pl.ds(i * _F, _F), pl.ds(fx * _F, _F), :],
            buf_a.at[s], in_sems[s])

    def wait_gather(s):
        pltpu.make_async_copy(
            x_hbm.at[0, pl.ds(0, _F), pl.ds(0, _F), :],
            buf_a.at[s], in_sems[s]).wait()

    def repack(s):
        for yi in range(_F):
            for xi in range(_F):
                for c in range(_C // _L):
                    buf_b[s, pl.ds(yi * _ROW + xi * _C + c * _L, _L)] = (
                        buf_a[s, yi, xi, pl.ds(c * _L, _L)])

    def fire_write(g, s):
        b, i, fx = unit_idx(g)
        pltpu.async_copy(buf_b.at[s], out_hbm.at[b, i, fx, :], out_sems[s])

    def wait_write(s):
        pltpu.make_async_copy(
            buf_b.at[s], out_hbm.at[0, 0, 0, :], out_sems[s]).wait()

    fire_gather(jnp.int32(0), 0)

    def iter_body(t, _):
        for s in range(_NS):
            g = t * _NS + s

            @pl.when(g + 1 < _UNITS)
            def _prefetch():
                fire_gather(g + 1, 1 - s)

            wait_gather(s)

            @pl.when(g >= _NS)
            def _recycle():
                wait_write(s)

            repack(s)
            fire_write(g, s)
        return ()

    lax.fori_loop(0, _UNITS // _NS, iter_body, ())
    for s in range(_NS):
        wait_write(s)


_fold = functools.partial(
    pl.kernel,
    mesh=plsc.VectorSubcoreMesh(core_axis_name="c", subcore_axis_name="s"),
    out_type=jax.ShapeDtypeStruct((_B, _WO, _HO, _C_OUT), jnp.float32),
    scratch_types=[
        pltpu.VMEM((_NS, _F, _F, _C), jnp.float32),
        pltpu.VMEM((_NS, _C_OUT), jnp.float32),
        pltpu.SemaphoreType.DMA,
        pltpu.SemaphoreType.DMA,
        pltpu.SemaphoreType.DMA,
        pltpu.SemaphoreType.DMA,
    ],
)(_fold_body)


def kernel(tensor):
    return _fold(tensor)
